# Initial kernel scaffold; baseline (speedup 1.0000x reference)
import jax
import jax.numpy as jnp
from jax.experimental import pallas as pl


def kernel(feature_vals, re_loc, intercept):
    raise NotImplementedError("write your pallas kernel here")


# speedup vs baseline: 43.0600x; 43.0600x over previous
import functools

import jax
import jax.numpy as jnp
from jax import lax
from jax.experimental import pallas as pl
from jax.experimental.pallas import tpu as pltpu
from jax.experimental.pallas import tpu_sc as plsc

NUM_LEVELS = 100000
TPAD = NUM_LEVELS + 8

NC = 2
NS = 16
L = 16
NW = NC * NS


@functools.lru_cache(maxsize=None)
def _build(B: int):
    assert B % (NW * L) == 0
    per_w = B // NW
    n_chunks = per_w // L

    mesh = plsc.VectorSubcoreMesh(core_axis_name="c", subcore_axis_name="s")

    @functools.partial(
        pl.kernel,
        mesh=mesh,
        compiler_params=pltpu.CompilerParams(needs_layout_passes=False),
        out_type=jax.ShapeDtypeStruct((B,), jnp.float32),
        scratch_types=[
            pltpu.VMEM((TPAD,), jnp.float32),
            pltpu.VMEM((per_w,), jnp.int32),
            pltpu.VMEM((per_w,), jnp.float32),
            pltpu.VMEM((L,), jnp.float32),
        ],
    )
    def sc_gather(fv_hbm, table_hbm, int_hbm, out_hbm, table_v, idx_v, out_v, int_v):
        wid = lax.axis_index("s") * NC + lax.axis_index("c")
        base = wid * per_w
        pltpu.sync_copy(table_hbm, table_v)
        pltpu.sync_copy(fv_hbm.at[pl.ds(base, per_w)], idx_v)
        pltpu.sync_copy(int_hbm, int_v)
        inter = int_v[...]

        def body(i, carry):
            s = pl.ds(i * L, L)
            idx = idx_v[s]
            valid = (idx >= 0) & (idx < NUM_LEVELS)
            idx2 = jnp.where(valid, idx, NUM_LEVELS)
            out_v[s] = plsc.load_gather(table_v, [idx2]) + inter
            return carry

        lax.fori_loop(0, n_chunks, body, 0, unroll=8)
        pltpu.sync_copy(out_v, out_hbm.at[pl.ds(base, per_w)])

    return sc_gather


def kernel(feature_vals, re_loc, intercept):
    shape = feature_vals.shape
    fv = feature_vals.reshape(-1).astype(jnp.int32)
    table = jnp.concatenate(
        [re_loc, jnp.zeros((TPAD - NUM_LEVELS,), re_loc.dtype)])
    ivec = jnp.full((L,), intercept, jnp.float32)
    out = _build(fv.size)(fv, table, ivec)
    return out.reshape(shape)


# speedup vs baseline: 49.1578x; 1.1416x over previous
import functools

import jax
import jax.numpy as jnp
from jax import lax
from jax.experimental import pallas as pl
from jax.experimental.pallas import tpu as pltpu
from jax.experimental.pallas import tpu_sc as plsc

NUM_LEVELS = 100000
TPAD = NUM_LEVELS + 8

NC = 2
NS = 16
L = 16
NW = NC * NS


@functools.lru_cache(maxsize=None)
def _build(B: int):
    assert B % (NW * L) == 0
    per_w = B // NW
    n_chunks = per_w // L

    mesh = plsc.VectorSubcoreMesh(core_axis_name="c", subcore_axis_name="s")

    @functools.partial(
        pl.kernel,
        mesh=mesh,
        compiler_params=pltpu.CompilerParams(needs_layout_passes=False),
        out_type=jax.ShapeDtypeStruct((B,), jnp.float32),
        scratch_types=[
            pltpu.VMEM((TPAD,), jnp.float32),
            pltpu.VMEM((per_w,), jnp.int32),
            pltpu.VMEM((per_w,), jnp.float32),
            pltpu.VMEM((L,), jnp.float32),
            pltpu.SemaphoreType.DMA,
        ],
    )
    def sc_gather(fv_hbm, table_hbm, int_hbm, out_hbm, table_v, idx_v, out_v, int_v, sem):
        wid = lax.axis_index("s") * NC + lax.axis_index("c")
        base = wid * per_w
        cp_t = pltpu.async_copy(table_hbm, table_v, sem)
        cp_i = pltpu.async_copy(fv_hbm.at[pl.ds(base, per_w)], idx_v, sem)
        cp_s = pltpu.async_copy(int_hbm, int_v, sem)
        cp_t.wait()
        cp_i.wait()
        cp_s.wait()
        inter = int_v[...]

        @plsc.parallel_loop(0, n_chunks, unroll=8)
        def body(i):
            s = pl.ds(i * L, L)
            idx = idx_v[s]
            valid = (idx >= 0) & (idx < NUM_LEVELS)
            idx2 = jnp.where(valid, idx, NUM_LEVELS)
            out_v[s] = plsc.load_gather(table_v, [idx2]) + inter

        pltpu.sync_copy(out_v, out_hbm.at[pl.ds(base, per_w)])

    return sc_gather


def kernel(feature_vals, re_loc, intercept):
    shape = feature_vals.shape
    fv = feature_vals.reshape(-1).astype(jnp.int32)
    table = jnp.concatenate(
        [re_loc, jnp.zeros((TPAD - NUM_LEVELS,), re_loc.dtype)])
    ivec = jnp.full((L,), intercept, jnp.float32)
    out = _build(fv.size)(fv, table, ivec)
    return out.reshape(shape)


# speedup vs baseline: 57.9359x; 1.1786x over previous
import functools

import jax
import jax.numpy as jnp
from jax import lax
from jax.experimental import pallas as pl
from jax.experimental.pallas import tpu as pltpu
from jax.experimental.pallas import tpu_sc as plsc

NUM_LEVELS = 100000
TPAD = NUM_LEVELS + 8

NC = 2
NS = 16
L = 16
NW = NC * NS


BR = 64


@functools.lru_cache(maxsize=None)
def _build(R: int, C: int):
    assert R % (NW * BR) == 0 and L <= C <= 2 * L
    rows_w = R // NW
    n_blk = rows_w // BR

    mesh = plsc.VectorSubcoreMesh(core_axis_name="c", subcore_axis_name="s")

    @functools.partial(
        pl.kernel,
        mesh=mesh,
        compiler_params=pltpu.CompilerParams(needs_layout_passes=False),
        out_type=jax.ShapeDtypeStruct((R, C), jnp.float32),
        scratch_types=[
            pltpu.VMEM((TPAD,), jnp.float32),
            pltpu.VMEM((BR, C), jnp.int32),
            pltpu.VMEM((BR, C), jnp.float32),
            pltpu.VMEM((L,), jnp.float32),
            pltpu.SemaphoreType.DMA,
        ],
    )
    def sc_gather(fv_hbm, table_hbm, int_hbm, out_hbm, table_v, idx_v, out_v, int_v, sem):
        wid = lax.axis_index("s") * NC + lax.axis_index("c")
        base = wid * rows_w
        cp_t = pltpu.async_copy(table_hbm, table_v, sem)
        cp_s = pltpu.async_copy(int_hbm, int_v, sem)
        cp_t.wait()
        cp_s.wait()
        inter = int_v[...]

        def blk(b, carry):
            r0 = base + b * BR
            pltpu.sync_copy(fv_hbm.at[pl.ds(r0, BR)], idx_v)

            @plsc.parallel_loop(0, BR, unroll=4)
            def body(r):
                for off in (0, C - L):
                    s = pl.ds(off, L)
                    idx = idx_v[r, s]
                    valid = (idx >= 0) & (idx < NUM_LEVELS)
                    idx2 = jnp.where(valid, idx, NUM_LEVELS)
                    out_v[r, s] = plsc.load_gather(table_v, [idx2]) + inter

            pltpu.sync_copy(out_v, out_hbm.at[pl.ds(r0, BR)])
            return carry

        lax.fori_loop(0, n_blk, blk, 0)

    return sc_gather


def kernel(feature_vals, re_loc, intercept):
    R, C = feature_vals.shape
    fv = feature_vals.astype(jnp.int32)
    table = jnp.concatenate(
        [re_loc, jnp.zeros((TPAD - NUM_LEVELS,), re_loc.dtype)])
    ivec = jnp.full((L,), intercept, jnp.float32)
    return _build(R, C)(fv, table, ivec)


# speedup vs baseline: 100.5437x; 1.7354x over previous
import functools

import jax
import jax.numpy as jnp
from jax import lax
from jax.experimental import pallas as pl
from jax.experimental.pallas import tpu as pltpu
from jax.experimental.pallas import tpu_sc as plsc

NUM_LEVELS = 100000
TPAD = NUM_LEVELS + 8

NC = 2
NS = 16
L = 16
NW = NC * NS
BC = 256


@functools.lru_cache(maxsize=None)
def _build(C: int, R: int):
    assert R % (NW * BC) == 0
    cols_w = R // NW
    n_blk = cols_w // BC

    mesh = plsc.VectorSubcoreMesh(core_axis_name="c", subcore_axis_name="s")

    @functools.partial(
        pl.kernel,
        mesh=mesh,
        compiler_params=pltpu.CompilerParams(needs_layout_passes=False),
        out_type=jax.ShapeDtypeStruct((C, R), jnp.float32),
        scratch_types=[
            pltpu.VMEM((TPAD,), jnp.float32),
            pltpu.VMEM((C, BC), jnp.int32),
            pltpu.VMEM((C, BC), jnp.int32),
            pltpu.VMEM((C, BC), jnp.float32),
            pltpu.VMEM((L,), jnp.float32),
            pltpu.SemaphoreType.DMA,
            pltpu.SemaphoreType.DMA,
        ],
    )
    def sc_gather(fv_hbm, table_hbm, int_hbm, out_hbm,
                  table_v, idx_a, idx_b, out_v, int_v, sem_t, sem_i):
        wid = lax.axis_index("s") * NC + lax.axis_index("c")
        base = wid * cols_w
        cp_t = pltpu.async_copy(table_hbm, table_v, sem_t)
        idx_bufs = (idx_a, idx_b)
        cps = [
            pltpu.async_copy(
                fv_hbm.at[:, pl.ds(base + b * BC, BC)], idx_bufs[b], sem_i)
            for b in range(n_blk)
        ]
        cp_s = pltpu.async_copy(int_hbm, int_v, sem_t)
        cp_t.wait()
        cp_s.wait()
        inter = int_v[...]

        for b in range(n_blk):
            cps[b].wait()
            idx_v = idx_bufs[b]

            @plsc.parallel_loop(0, C * (BC // L), unroll=4)
            def body(i):
                r = i // (BC // L)
                s = pl.ds((i % (BC // L)) * L, L)
                idx = idx_v[r, s]
                valid = (idx >= 0) & (idx < NUM_LEVELS)
                idx2 = jnp.where(valid, idx, NUM_LEVELS)
                out_v[r, s] = plsc.load_gather(table_v, [idx2]) + inter

            pltpu.sync_copy(out_v, out_hbm.at[:, pl.ds(base + b * BC, BC)])

    return sc_gather


def kernel(feature_vals, re_loc, intercept):
    R, C = feature_vals.shape
    fvT = feature_vals.astype(jnp.int32).T
    table = jnp.concatenate(
        [re_loc, jnp.zeros((TPAD - NUM_LEVELS,), re_loc.dtype)])
    ivec = jnp.full((L,), intercept, jnp.float32)
    outT = _build(C, R)(fvT, table, ivec)
    return outT.T


# speedup vs baseline: 124.6223x; 1.2395x over previous
import functools

import jax
import jax.numpy as jnp
from jax import lax
from jax.experimental import pallas as pl
from jax.experimental.pallas import tpu as pltpu
from jax.experimental.pallas import tpu_sc as plsc

NUM_LEVELS = 100000
TPAD = NUM_LEVELS + 16

NC = 2
NS = 16
L = 16
NW = NC * NS
BC = 256


@functools.lru_cache(maxsize=None)
def _build(C: int, R: int):
    assert R % (NW * BC) == 0
    cols_w = R // NW
    n_blk = cols_w // BC

    mesh = plsc.VectorSubcoreMesh(core_axis_name="c", subcore_axis_name="s")

    @functools.partial(
        pl.kernel,
        mesh=mesh,
        compiler_params=pltpu.CompilerParams(needs_layout_passes=False),
        out_type=jax.ShapeDtypeStruct((C, R), jnp.float32),
        scratch_types=[
            pltpu.VMEM_SHARED((NUM_LEVELS,), jnp.float32),
            pltpu.VMEM((TPAD,), jnp.float32),
            pltpu.VMEM((C, BC), jnp.int32),
            pltpu.VMEM((C, BC), jnp.int32),
            pltpu.VMEM((C, BC), jnp.float32),
            pltpu.VMEM((L,), jnp.float32),
            pltpu.SemaphoreType.DMA,
            pltpu.SemaphoreType.DMA,
        ],
    )
    def sc_gather(fv_hbm, table_hbm, int_hbm, out_hbm,
                  table_sh, table_v, idx_a, idx_b, out_v, int_v, sem_t, sem_i):
        cid = lax.axis_index("c")
        sid = lax.axis_index("s")
        wid = sid * NC + cid
        base = wid * cols_w
        idx_bufs = (idx_a, idx_b)
        cps = [
            pltpu.async_copy(
                fv_hbm.at[:, pl.ds(base + b * BC, BC)], idx_bufs[b], sem_i)
            for b in range(n_blk)
        ]
        cp_s = pltpu.async_copy(int_hbm, int_v, sem_t)

        @pl.when(sid == 0)
        def _():
            pltpu.sync_copy(table_hbm, table_sh)

        plsc.subcore_barrier()
        pltpu.sync_copy(table_sh, table_v.at[pl.ds(0, NUM_LEVELS)])
        table_v[pl.ds(NUM_LEVELS, L)] = jnp.zeros((L,), jnp.float32)
        cp_s.wait()
        inter = int_v[...]

        for b in range(n_blk):
            cps[b].wait()
            idx_v = idx_bufs[b]

            @plsc.parallel_loop(0, C * (BC // L), unroll=4)
            def body(i):
                r = i // (BC // L)
                s = pl.ds((i % (BC // L)) * L, L)
                idx = idx_v[r, s]
                valid = (idx >= 0) & (idx < NUM_LEVELS)
                idx2 = jnp.where(valid, idx, NUM_LEVELS)
                out_v[r, s] = plsc.load_gather(table_v, [idx2]) + inter

            pltpu.sync_copy(out_v, out_hbm.at[:, pl.ds(base + b * BC, BC)])

    return sc_gather


def kernel(feature_vals, re_loc, intercept):
    R, C = feature_vals.shape
    fvT = feature_vals.astype(jnp.int32).T
    ivec = jnp.full((L,), intercept, jnp.float32)
    outT = _build(C, R)(fvT, re_loc, ivec)
    return outT.T
